# Initial kernel scaffold; baseline (speedup 1.0000x reference)
import jax
import jax.numpy as jnp
from jax.experimental import pallas as pl


def kernel(student_features, teacher_codes, codebook):
    raise NotImplementedError("write your pallas kernel here")


# speedup vs baseline: 14.6610x; 14.6610x over previous
import functools

import jax
import jax.numpy as jnp
from jax.experimental import pallas as pl
from jax.experimental.pallas import tpu as pltpu

_TEMPERATURE = 0.1
_NUM_NEGATIVES = 16
_TN = 256


def _ctl_kernel(s_ref, tc_ref, cb_ref, out_ref, *, n_total):
    i = pl.program_id(0)
    s = s_ref[...]
    cb = cb_ref[...]
    tc = tc_ref[...]
    tn, d_dim = s.shape
    k_dim = cb.shape[0]

    cb_sq = jnp.sum(cb * cb, axis=1, keepdims=True).T
    s_sq = jnp.sum(s * s, axis=1, keepdims=True)
    cross = jax.lax.dot_general(
        s, cb, (((1,), (1,)), ((), ())),
        preferred_element_type=jnp.float32)

    sq = s_sq + cb_sq - 2.0 * cross
    dist = jnp.maximum(sq, 0.0)
    col = jax.lax.broadcasted_iota(jnp.int32, (tn, k_dim), 1)
    posmask = col == tc
    inf = jnp.float32(jnp.inf)
    dist = jnp.where(posmask, inf, dist)

    work = dist
    m = jnp.min(work, axis=1, keepdims=True)
    for _ in range(_NUM_NEGATIVES - 1):
        work = jnp.where(work == m, inf, work)
        m = jnp.min(work, axis=1, keepdims=True)
    theta = m

    selmask = dist <= theta

    s_n = jnp.maximum(jnp.sqrt(s_sq), 1e-12)
    cb_n = jnp.maximum(jnp.sqrt(cb_sq), 1e-12)
    sim = cross / (s_n * cb_n)

    z_neg = jnp.sum(
        jnp.where(selmask, jnp.exp(sim / _TEMPERATURE), 0.0),
        axis=1, keepdims=True)
    pos_sim = jnp.sum(jnp.where(posmask, sim, 0.0), axis=1,
                      keepdims=True) / _TEMPERATURE
    ce = jnp.log(jnp.exp(pos_sim) + z_neg) - pos_sim
    tile_loss = jnp.sum(ce, axis=0, keepdims=True) / n_total

    @pl.when(i == 0)
    def _():
        out_ref[...] = jnp.zeros_like(out_ref)

    out_ref[...] += tile_loss


def kernel(student_features, teacher_codes, codebook):
    b, t, d_dim = student_features.shape
    n = b * t
    k_dim = codebook.shape[0]
    s_flat = student_features.reshape(n, d_dim)
    tc_flat = teacher_codes.reshape(n, 1).astype(jnp.int32)
    num_tiles = n // _TN

    out = pl.pallas_call(
        functools.partial(_ctl_kernel, n_total=n),
        grid=(num_tiles,),
        in_specs=[
            pl.BlockSpec((_TN, d_dim), lambda i: (i, 0)),
            pl.BlockSpec((_TN, 1), lambda i: (i, 0)),
            pl.BlockSpec((k_dim, d_dim), lambda i: (0, 0)),
        ],
        out_specs=pl.BlockSpec((1, 1), lambda i: (0, 0)),
        out_shape=jax.ShapeDtypeStruct((1, 1), jnp.float32),
        compiler_params=pltpu.CompilerParams(
            dimension_semantics=("arbitrary",)),
    )(s_flat, tc_flat, codebook)
    return out[0, 0]
